# baseline (speedup 1.0000x reference)
import functools

import jax
import jax.numpy as jnp
from jax import lax
from jax.experimental import pallas as pl
from jax.experimental.pallas import tpu as pltpu
from jax.experimental.pallas import tpu_sc as plsc

NC = 2
NS = 16
NW = NC * NS
W = 256
LAGG = 8


@functools.lru_cache(maxsize=None)
def _build(V, D, B):
    assert B % (NW * 16) == 0 and D % 16 == 0 and V >= W
    b_per_w = B // NW
    n_grp = b_per_w // 16
    mesh = plsc.VectorSubcoreMesh(core_axis_name="c", subcore_axis_name="s")

    @functools.partial(
        pl.kernel,
        out_type=jax.ShapeDtypeStruct((B * D,), jnp.float32),
        mesh=mesh,
        scratch_types=[
            pltpu.VMEM((n_grp, 16), jnp.int32),
            pltpu.VMEM((W * D,), jnp.float32),
            pltpu.VMEM((16 * D,), jnp.float32),
            pltpu.SemaphoreType.DMA,
        ],
    )
    def gather_kernel(table_hbm, idx_hbm, out_hbm, idx_v, win, stage, sem):
        wid = lax.axis_index("s") * NC + lax.axis_index("c")
        pltpu.sync_copy(idx_hbm.at[wid], idx_v)
        base = wid * b_per_w
        wmin = idx_v[0, pl.ds(0, 16)][0]
        wmax = idx_v[n_grp - 1, pl.ds(0, 16)][15]
        wstart = jnp.maximum(jnp.minimum(wmin, V - W), 0)

        def m8(x):
            return pl.multiple_of(x, 8)

        def drain16():
            for _ in range(16):
                pltpu.make_async_copy(
                    win.at[pl.ds(0, D)], out_hbm.at[pl.ds(0, D)],
                    sem).wait()

        @pl.when(wmax - wstart < W)
        def _fast():
            pltpu.sync_copy(
                table_hbm.at[pl.ds(m8(wstart * D), W * D)], win)

            def grp_body(grp, carry):
                pvec = (idx_v[grp, pl.ds(0, 16)] - wstart) * D
                abase = (base + grp * 16) * D
                for l in range(16):
                    pltpu.async_copy(
                        win.at[pl.ds(m8(pvec[l]), D)],
                        out_hbm.at[pl.ds(m8(abase + l * D), D)], sem)

                @pl.when(grp >= LAGG)
                def _():
                    drain16()

                return carry

            lax.fori_loop(0, n_grp, grp_body, 0)

            def tail_body(i, carry):
                drain16()
                return carry

            lax.fori_loop(0, min(LAGG, n_grp), tail_body, 0)

        @pl.when(wmax - wstart >= W)
        def _general():
            def fb_group(grp, carry):
                pvec = idx_v[grp, pl.ds(0, 16)] * D
                for l in range(16):
                    pltpu.async_copy(
                        table_hbm.at[pl.ds(m8(pvec[l]), D)],
                        stage.at[pl.ds(l * D, D)], sem)
                for l in range(16):
                    pltpu.make_async_copy(
                        table_hbm.at[pl.ds(0, D)],
                        stage.at[pl.ds(0, D)], sem).wait()
                pltpu.sync_copy(
                    stage,
                    out_hbm.at[pl.ds(m8((base + grp * 16) * D), 16 * D)])
                return carry

            lax.fori_loop(0, n_grp, fb_group, 0)

    return gather_kernel


def kernel(system_features, batch_index):
    V, D = system_features.shape
    (B,) = batch_index.shape
    idx = batch_index.astype(jnp.int32).reshape(NW, B // (NW * 16), 16)
    out = _build(V, D, B)(system_features.reshape(-1), idx)
    return out.reshape(B, D)
